# baseline (speedup 1.0000x reference)
import functools

import jax
import jax.numpy as jnp
from jax import lax
from jax.experimental import pallas as pl
from jax.experimental.pallas import tpu as pltpu
from jax.experimental.pallas import tpu_sc as plsc

TOPK = 2
E = 8
D = 2048
F = 1024
T = 2048
BT = 128
NBLK = T * TOPK // BT + E
S = NBLK * BT
LANES = 128
NEG = -1e30


def _router_dispatch_body(x_ref, gw_ref, bias_ref, sw_ref, rw_ref, cnt_ref,
                          xcopy_ref):
    x = x_ref[...]
    xcopy_ref[...] = x
    gw = gw_ref[...]
    logits = lax.dot_general(x, gw, (((1,), (1,)), ((), ())),
                             preferred_element_type=jnp.float32)
    lane = lax.broadcasted_iota(jnp.int32, (T, LANES), 1)
    valid = lane < E
    rw = jax.nn.sigmoid(logits)
    score = jnp.where(valid, rw + bias_ref[...], NEG)

    m0 = jnp.max(score, axis=1, keepdims=True)
    i0 = jnp.min(jnp.where((score == m0) & valid, lane, LANES), axis=1,
                 keepdims=True)
    oh0 = (lane == i0)
    score1 = jnp.where(oh0, NEG, score)
    m1 = jnp.max(score1, axis=1, keepdims=True)
    i1 = jnp.min(jnp.where((score1 == m1) & valid, lane, LANES), axis=1,
                 keepdims=True)
    oh1 = (lane == i1)

    w0 = jnp.sum(jnp.where(oh0, rw, 0.0), axis=1, keepdims=True)
    w1 = jnp.sum(jnp.where(oh1, rw, 0.0), axis=1, keepdims=True)
    tot = w0 + w1
    w0n = w0 / tot
    w1n = w1 / tot

    oh0f = oh0.astype(jnp.float32)
    oh1f = oh1.astype(jnp.float32)

    row = lax.broadcasted_iota(jnp.int32, (T, LANES), 0)
    c0, c1 = oh0f, oh1f
    k = 1
    while k < T:
        c0 = c0 + jnp.where(row >= k, pltpu.roll(c0, k, 0), 0.0)
        c1 = c1 + jnp.where(row >= k, pltpu.roll(c1, k, 0), 0.0)
        k *= 2

    cnt0 = c0[T - 1:T, :]
    cnt1 = c1[T - 1:T, :]
    cnt = cnt0 + cnt1
    cnt_ref[...] = cnt

    r0 = jnp.sum(oh0f * c0, axis=1, keepdims=True) - 1.0
    r1 = (jnp.sum(oh1f * c1, axis=1, keepdims=True) - 1.0
          + jnp.sum(oh1f * cnt0, axis=1, keepdims=True))

    pcnt = jnp.floor((cnt + (BT - 1)) * (1.0 / BT)).astype(jnp.float32) * BT
    r_i = lax.broadcasted_iota(jnp.int32, (LANES, LANES), 0)
    c_i = lax.broadcasted_iota(jnp.int32, (LANES, LANES), 1)
    tri = (r_i < c_i).astype(jnp.float32)
    pstart = lax.dot_general(pcnt, tri, (((1,), (0,)), ((), ())),
                             preferred_element_type=jnp.float32)

    slot0 = jnp.sum(oh0f * pstart, axis=1, keepdims=True) + r0
    slot1 = jnp.sum(oh1f * pstart, axis=1, keepdims=True) + r1
    sw_ref[...] = jnp.concatenate([slot0, slot1, w0n, w1n], axis=1)

    slot8 = jnp.concatenate(
        [slot0, slot1] + [jnp.full((T, 1), -1.0, jnp.float32)] * 6, axis=1)
    slot_row = jnp.transpose(slot8, (1, 0))[0:2, :].reshape(1, 2 * T)
    a_col = lax.broadcasted_iota(jnp.int32, (2 * T, 1), 0)
    tok_col = (a_col % T).astype(jnp.float32)
    w_col = jnp.concatenate([w0n, w1n], axis=0)
    rhs = jnp.concatenate(
        [tok_col, w_col] + [jnp.zeros((2 * T, 1), jnp.float32)] * 6, axis=1)
    for b in range(NBLK):
        srange = (b * BT
                  + lax.broadcasted_iota(jnp.int32, (BT, 1), 0).astype(
                      jnp.float32))
        cmp = (slot_row == srange).astype(jnp.float32)
        res = lax.dot_general(cmp, rhs, (((1,), (0,)), ((), ())),
                              preferred_element_type=jnp.float32)
        rw_ref[pl.ds(b * BT, BT), :] = res


def _router_dispatch(x, gate_w, bias):
    gw = jnp.zeros((LANES, D), jnp.float32).at[:E, :].set(gate_w)
    bias_p = jnp.zeros((1, LANES), jnp.float32).at[0, :E].set(bias)
    return pl.pallas_call(
        _router_dispatch_body,
        out_shape=(
            jax.ShapeDtypeStruct((T, 4), jnp.float32),
            jax.ShapeDtypeStruct((S, 8), jnp.float32),
            jax.ShapeDtypeStruct((1, LANES), jnp.float32),
            jax.ShapeDtypeStruct((T, D), jnp.float32),
        ),
    )(x, gw, bias_p)


def _ffn_body(be_ref, nb_ref, xs_ref, wt_ref, w1_ref, w2_ref, w3_ref, ys_ref):
    b = pl.program_id(0)

    @pl.when(b < nb_ref[0])
    def _():
        x = xs_ref[...]
        h = lax.dot_general(x, w1_ref[0], (((1,), (1,)), ((), ())),
                            preferred_element_type=jnp.float32)
        g = lax.dot_general(x, w3_ref[0], (((1,), (1,)), ((), ())),
                            preferred_element_type=jnp.float32)
        u = (h * jax.nn.sigmoid(h)) * g * wt_ref[...]
        ys_ref[...] = lax.dot_general(u, w2_ref[0], (((1,), (1,)), ((), ())),
                                      preferred_element_type=jnp.float32)


def _ffn(xs, wt_col, w1, w2, w3, block_expert, nblk):
    grid_spec = pltpu.PrefetchScalarGridSpec(
        num_scalar_prefetch=2,
        grid=(NBLK,),
        in_specs=[
            pl.BlockSpec((BT, D), lambda b, be, nb: (b, 0)),
            pl.BlockSpec((BT, 1), lambda b, be, nb: (b, 0)),
            pl.BlockSpec((1, F, D), lambda b, be, nb: (be[b], 0, 0)),
            pl.BlockSpec((1, D, F), lambda b, be, nb: (be[b], 0, 0)),
            pl.BlockSpec((1, F, D), lambda b, be, nb: (be[b], 0, 0)),
        ],
        out_specs=pl.BlockSpec((BT, D), lambda b, be, nb: (b, 0)),
    )
    return pl.pallas_call(
        _ffn_body,
        grid_spec=grid_spec,
        out_shape=jax.ShapeDtypeStruct((S, D), jnp.float32),
    )(block_expert, nblk, xs, wt_col, w1, w2, w3)


def _sc_mesh():
    return plsc.VectorSubcoreMesh(core_axis_name="c", subcore_axis_name="s")


def _sc_gather(x, row_ids):
    info = plsc.get_sparse_core_info()
    nw = info.num_cores * info.num_subcores
    rows_per_w = S // nw
    ch = 8
    nbuf = 7
    nch = rows_per_w // ch

    @functools.partial(
        pl.kernel,
        out_type=jax.ShapeDtypeStruct((S, D), jnp.float32),
        mesh=_sc_mesh(),
        scratch_types=(
            [pltpu.VMEM((rows_per_w,), jnp.int32)]
            + [pltpu.VMEM((ch, D), jnp.float32) for _ in range(nbuf)]
            + [pltpu.SemaphoreType.DMA for _ in range(2 * nbuf)]
        ),
    )
    def k(x_hbm, ids_hbm, xs_hbm, idx_v, *bufs_sems):
        rows_v = bufs_sems[:nbuf]
        sem_g = bufs_sems[nbuf:2 * nbuf]
        sem_w = bufs_sems[2 * nbuf:]
        wid = lax.axis_index("s") * info.num_cores + lax.axis_index("c")
        base = wid * rows_per_w
        pltpu.sync_copy(ids_hbm.at[pl.ds(base, rows_per_w)], idx_v)
        look = nbuf - 1
        descs_g = [None] * nbuf
        descs_w = [None] * nbuf
        for c in range(nch + look):
            if c < nch:
                b = c % nbuf
                if descs_w[b] is not None:
                    descs_w[b].wait()
                    descs_w[b] = None
                descs_g[b] = pltpu.async_copy(
                    x_hbm.at[idx_v.at[pl.ds(c * ch, ch)]], rows_v[b],
                    sem_g[b])
            if c >= look:
                cp = c - look
                bp = cp % nbuf
                descs_g[bp].wait()
                descs_w[bp] = pltpu.async_copy(
                    rows_v[bp], xs_hbm.at[pl.ds(base + cp * ch, ch)],
                    sem_w[bp])
        for b in range(nbuf):
            if descs_w[b] is not None:
                descs_w[b].wait()

    return k(x, row_ids)


def _sc_combine(ys, slot0, slot1):
    info = plsc.get_sparse_core_info()
    nw = info.num_cores * info.num_subcores
    rows_per_w = T // nw
    ch = 8
    nbuf = 6
    lag = 2
    nch = rows_per_w // ch

    @functools.partial(
        pl.kernel,
        out_type=jax.ShapeDtypeStruct((T, D), jnp.float32),
        mesh=_sc_mesh(),
        scratch_types=(
            [pltpu.VMEM((rows_per_w,), jnp.int32),
             pltpu.VMEM((rows_per_w,), jnp.int32)]
            + [pltpu.VMEM((ch, D), jnp.float32) for _ in range(nbuf)]
            + [pltpu.SemaphoreType.DMA for _ in range(3 * nbuf)]
        ),
    )
    def k(ys_hbm, s0_hbm, s1_hbm, out_hbm, idx0_v, idx1_v, *bufs_sems):
        rows_v = bufs_sems[:nbuf]
        sem_g0 = bufs_sems[nbuf:2 * nbuf]
        sem_g1 = bufs_sems[2 * nbuf:3 * nbuf]
        sem_w = bufs_sems[3 * nbuf:]
        wid = lax.axis_index("s") * info.num_cores + lax.axis_index("c")
        base = wid * rows_per_w
        pltpu.sync_copy(s0_hbm.at[pl.ds(base, rows_per_w)], idx0_v)
        pltpu.sync_copy(s1_hbm.at[pl.ds(base, rows_per_w)], idx1_v)
        descs_g0 = [None] * nbuf
        descs_g1 = [None] * nbuf
        descs_w = [None] * nbuf
        for c in range(nch + 2 * lag):
            if c < nch:
                b = c % nbuf
                if descs_w[b] is not None:
                    descs_w[b].wait()
                    descs_w[b] = None
                descs_g0[b] = pltpu.async_copy(
                    ys_hbm.at[idx0_v.at[pl.ds(c * ch, ch)]], rows_v[b],
                    sem_g0[b])
            if lag <= c < nch + lag:
                cp = c - lag
                bp = cp % nbuf
                descs_g0[bp].wait()
                descs_g1[bp] = pltpu.async_copy(
                    ys_hbm.at[idx1_v.at[pl.ds(cp * ch, ch)]], rows_v[bp],
                    sem_g1[bp], add=True)
            if c >= 2 * lag:
                cq = c - 2 * lag
                bq = cq % nbuf
                descs_g1[bq].wait()
                descs_w[bq] = pltpu.async_copy(
                    rows_v[bq], out_hbm.at[pl.ds(base + cq * ch, ch)],
                    sem_w[bq])
        for b in range(nbuf):
            if descs_w[b] is not None:
                descs_w[b].wait()

    return k(ys, slot0, slot1)


def kernel(hidden_states, gate_w, e_score_correction_bias, w1, w2, w3):
    bsz, seq_len, hidden_dim = hidden_states.shape
    x = hidden_states.reshape(T, D)

    sw, rowwt, cnt, xcopy = _router_dispatch(x, gate_w,
                                             e_score_correction_bias)

    slot0 = sw[:, 0].astype(jnp.int32)
    slot1 = sw[:, 1].astype(jnp.int32)
    row_ids = rowwt[:, 0].astype(jnp.int32)
    wt_col = rowwt[:, 1:2]

    cnt_e = cnt[0, :E].astype(jnp.int32)
    nblk_e = (cnt_e + BT - 1) // BT
    ends = jnp.cumsum(nblk_e)
    nblk_total = ends[E - 1:E]
    bidx = jnp.arange(NBLK, dtype=jnp.int32)
    block_expert = jnp.minimum(
        jnp.sum((bidx[:, None] >= ends[None, :]).astype(jnp.int32), axis=1),
        E - 1)

    xs = _sc_gather(xcopy, row_ids)

    ys = _ffn(xs, wt_col, w1, w2, w3, block_expert, nblk_total)

    out = _sc_combine(ys, slot0, slot1)

    return out.reshape(bsz, seq_len, hidden_dim)


# speedup vs baseline: 1.3384x; 1.3384x over previous
import functools

import jax
import jax.numpy as jnp
from jax import lax
from jax.experimental import pallas as pl
from jax.experimental.pallas import tpu as pltpu
from jax.experimental.pallas import tpu_sc as plsc

TOPK = 2
E = 8
D = 2048
F = 1024
T = 2048
BT = 128
NBLK = T * TOPK // BT + E
S = NBLK * BT
LANES = 128
NEG = -1e30


def _router_dispatch_body(x_ref, gw_ref, bias_ref, sw_ref, rw_ref, cnt_ref):
    x = x_ref[...]
    gw = gw_ref[...]
    logits = lax.dot_general(x, gw, (((1,), (1,)), ((), ())),
                             preferred_element_type=jnp.float32)
    lane = lax.broadcasted_iota(jnp.int32, (T, LANES), 1)
    valid = lane < E
    rw = jax.nn.sigmoid(logits)
    score = jnp.where(valid, rw + bias_ref[...], NEG)

    m0 = jnp.max(score, axis=1, keepdims=True)
    i0 = jnp.min(jnp.where((score == m0) & valid, lane, LANES), axis=1,
                 keepdims=True)
    oh0 = (lane == i0)
    score1 = jnp.where(oh0, NEG, score)
    m1 = jnp.max(score1, axis=1, keepdims=True)
    i1 = jnp.min(jnp.where((score1 == m1) & valid, lane, LANES), axis=1,
                 keepdims=True)
    oh1 = (lane == i1)

    w0 = jnp.sum(jnp.where(oh0, rw, 0.0), axis=1, keepdims=True)
    w1 = jnp.sum(jnp.where(oh1, rw, 0.0), axis=1, keepdims=True)
    tot = w0 + w1
    w0n = w0 / tot
    w1n = w1 / tot

    oh0f = oh0.astype(jnp.float32)
    oh1f = oh1.astype(jnp.float32)

    row = lax.broadcasted_iota(jnp.int32, (T, LANES), 0)
    c0, c1 = oh0f, oh1f
    k = 1
    while k < T:
        c0 = c0 + jnp.where(row >= k, pltpu.roll(c0, k, 0), 0.0)
        c1 = c1 + jnp.where(row >= k, pltpu.roll(c1, k, 0), 0.0)
        k *= 2

    cnt0 = c0[T - 1:T, :]
    cnt1 = c1[T - 1:T, :]
    cnt = cnt0 + cnt1
    cnt_ref[...] = cnt

    r0 = jnp.sum(oh0f * c0, axis=1, keepdims=True) - 1.0
    r1 = (jnp.sum(oh1f * c1, axis=1, keepdims=True) - 1.0
          + jnp.sum(oh1f * cnt0, axis=1, keepdims=True))

    pcnt = jnp.floor((cnt + (BT - 1)) * (1.0 / BT)).astype(jnp.float32) * BT
    r_i = lax.broadcasted_iota(jnp.int32, (LANES, LANES), 0)
    c_i = lax.broadcasted_iota(jnp.int32, (LANES, LANES), 1)
    tri = (r_i < c_i).astype(jnp.float32)
    pstart = lax.dot_general(pcnt, tri, (((1,), (0,)), ((), ())),
                             preferred_element_type=jnp.float32)

    slot0 = jnp.sum(oh0f * pstart, axis=1, keepdims=True) + r0
    slot1 = jnp.sum(oh1f * pstart, axis=1, keepdims=True) + r1
    sw_ref[...] = jnp.concatenate([slot0, slot1, w0n, w1n], axis=1)

    slot8 = jnp.concatenate(
        [slot0, slot1] + [jnp.full((T, 1), -1.0, jnp.float32)] * 6, axis=1)
    slot_row = jnp.transpose(slot8, (1, 0))[0:2, :].reshape(1, 2 * T)
    a_col = lax.broadcasted_iota(jnp.int32, (2 * T, 1), 0)
    tok_col = (a_col % T).astype(jnp.float32)
    w_col = jnp.concatenate([w0n, w1n], axis=0)
    rhs = jnp.concatenate(
        [tok_col, w_col] + [jnp.zeros((2 * T, 1), jnp.float32)] * 6, axis=1)
    for b in range(NBLK):
        srange = (b * BT
                  + lax.broadcasted_iota(jnp.int32, (BT, 1), 0).astype(
                      jnp.float32))
        cmp = (slot_row == srange).astype(jnp.float32)
        res = lax.dot_general(cmp, rhs, (((1,), (0,)), ((), ())),
                              preferred_element_type=jnp.float32)
        rw_ref[pl.ds(b * BT, BT), :] = res


def _router_dispatch(x, gate_w, bias):
    gw = jnp.zeros((LANES, D), jnp.float32).at[:E, :].set(gate_w)
    bias_p = jnp.zeros((1, LANES), jnp.float32).at[0, :E].set(bias)
    return pl.pallas_call(
        _router_dispatch_body,
        out_shape=(
            jax.ShapeDtypeStruct((T, 4), jnp.float32),
            jax.ShapeDtypeStruct((S, 8), jnp.float32),
            jax.ShapeDtypeStruct((1, LANES), jnp.float32),
        ),
    )(x, gw, bias_p)


def _ffn_body(be_ref, nb_ref, xs_ref, wt_ref, w1_ref, w2_ref, w3_ref, ys_ref):
    b = pl.program_id(0)

    @pl.when(b < nb_ref[0])
    def _():
        x = xs_ref[...]
        h = lax.dot_general(x, w1_ref[0], (((1,), (1,)), ((), ())),
                            preferred_element_type=jnp.float32)
        g = lax.dot_general(x, w3_ref[0], (((1,), (1,)), ((), ())),
                            preferred_element_type=jnp.float32)
        u = (h * jax.nn.sigmoid(h)) * g * wt_ref[...]
        ys_ref[...] = lax.dot_general(u, w2_ref[0], (((1,), (1,)), ((), ())),
                                      preferred_element_type=jnp.float32)


def _ffn(xs, wt_col, w1, w2, w3, block_expert, nblk):
    grid_spec = pltpu.PrefetchScalarGridSpec(
        num_scalar_prefetch=2,
        grid=(NBLK,),
        in_specs=[
            pl.BlockSpec((BT, D), lambda b, be, nb: (b, 0)),
            pl.BlockSpec((BT, 1), lambda b, be, nb: (b, 0)),
            pl.BlockSpec((1, F, D), lambda b, be, nb: (be[b], 0, 0)),
            pl.BlockSpec((1, D, F), lambda b, be, nb: (be[b], 0, 0)),
            pl.BlockSpec((1, F, D), lambda b, be, nb: (be[b], 0, 0)),
        ],
        out_specs=pl.BlockSpec((BT, D), lambda b, be, nb: (b, 0)),
    )
    return pl.pallas_call(
        _ffn_body,
        grid_spec=grid_spec,
        out_shape=jax.ShapeDtypeStruct((S, D), jnp.float32),
    )(block_expert, nblk, xs, wt_col, w1, w2, w3)


def _sc_mesh():
    return plsc.VectorSubcoreMesh(core_axis_name="c", subcore_axis_name="s")


def _sc_dispatch(x, slot2):
    info = plsc.get_sparse_core_info()
    nw = info.num_cores * info.num_subcores
    rows_per_w = T // nw
    ch = 8
    nbuf = 4
    lag = 2
    nch = rows_per_w // ch
    idx3 = slot2.reshape(nw, nch, ch, 2).transpose(0, 3, 1, 2).reshape(
        nw, 2 * nch, ch)

    @functools.partial(
        pl.kernel,
        out_type=jax.ShapeDtypeStruct((S, D), jnp.float32),
        mesh=_sc_mesh(),
        scratch_types=(
            [pltpu.VMEM((2 * nch, ch), jnp.int32)]
            + [pltpu.VMEM((ch, D), jnp.float32) for _ in range(nbuf)]
            + [pltpu.SemaphoreType.DMA for _ in range(3 * nbuf)]
        ),
    )
    def k(x_hbm, idx_hbm, xs_hbm, idx_v, *bufs_sems):
        rows_v = bufs_sems[:nbuf]
        sem_g = bufs_sems[nbuf:2 * nbuf]
        sem_s0 = bufs_sems[2 * nbuf:3 * nbuf]
        sem_s1 = bufs_sems[3 * nbuf:]
        wid = lax.axis_index("s") * info.num_cores + lax.axis_index("c")
        base = wid * rows_per_w
        pltpu.sync_copy(idx_hbm.at[wid], idx_v)
        descs_g = [None] * nbuf
        descs_s0 = [None] * nbuf
        descs_s1 = [None] * nbuf
        for c in range(nch + lag):
            if c < nch:
                b = c % nbuf
                if descs_s1[b] is not None:
                    descs_s0[b].wait()
                    descs_s1[b].wait()
                    descs_s0[b] = None
                    descs_s1[b] = None
                descs_g[b] = pltpu.async_copy(
                    x_hbm.at[pl.ds(base + c * ch, ch)], rows_v[b], sem_g[b])
            if c >= lag:
                cp = c - lag
                bp = cp % nbuf
                descs_g[bp].wait()
                descs_s0[bp] = pltpu.async_copy(
                    rows_v[bp], xs_hbm.at[idx_v.at[cp]], sem_s0[bp])
                descs_s1[bp] = pltpu.async_copy(
                    rows_v[bp], xs_hbm.at[idx_v.at[nch + cp]], sem_s1[bp])
        for b in range(nbuf):
            if descs_s1[b] is not None:
                descs_s0[b].wait()
                descs_s1[b].wait()

    return k(x, idx3)


def _sc_combine(ys, slot0, slot1):
    info = plsc.get_sparse_core_info()
    nw = info.num_cores * info.num_subcores
    rows_per_w = T // nw
    ch = 8
    nbuf = 6
    lag = 2
    nch = rows_per_w // ch

    @functools.partial(
        pl.kernel,
        out_type=jax.ShapeDtypeStruct((T, D), jnp.float32),
        mesh=_sc_mesh(),
        scratch_types=(
            [pltpu.VMEM((rows_per_w,), jnp.int32),
             pltpu.VMEM((rows_per_w,), jnp.int32)]
            + [pltpu.VMEM((ch, D), jnp.float32) for _ in range(nbuf)]
            + [pltpu.SemaphoreType.DMA for _ in range(3 * nbuf)]
        ),
    )
    def k(ys_hbm, s0_hbm, s1_hbm, out_hbm, idx0_v, idx1_v, *bufs_sems):
        rows_v = bufs_sems[:nbuf]
        sem_g0 = bufs_sems[nbuf:2 * nbuf]
        sem_g1 = bufs_sems[2 * nbuf:3 * nbuf]
        sem_w = bufs_sems[3 * nbuf:]
        wid = lax.axis_index("s") * info.num_cores + lax.axis_index("c")
        base = wid * rows_per_w
        pltpu.sync_copy(s0_hbm.at[pl.ds(base, rows_per_w)], idx0_v)
        pltpu.sync_copy(s1_hbm.at[pl.ds(base, rows_per_w)], idx1_v)
        descs_g0 = [None] * nbuf
        descs_g1 = [None] * nbuf
        descs_w = [None] * nbuf
        for c in range(nch + 2 * lag):
            if c < nch:
                b = c % nbuf
                if descs_w[b] is not None:
                    descs_w[b].wait()
                    descs_w[b] = None
                descs_g0[b] = pltpu.async_copy(
                    ys_hbm.at[idx0_v.at[pl.ds(c * ch, ch)]], rows_v[b],
                    sem_g0[b])
            if lag <= c < nch + lag:
                cp = c - lag
                bp = cp % nbuf
                descs_g0[bp].wait()
                descs_g1[bp] = pltpu.async_copy(
                    ys_hbm.at[idx1_v.at[pl.ds(cp * ch, ch)]], rows_v[bp],
                    sem_g1[bp], add=True)
            if c >= 2 * lag:
                cq = c - 2 * lag
                bq = cq % nbuf
                descs_g1[bq].wait()
                descs_w[bq] = pltpu.async_copy(
                    rows_v[bq], out_hbm.at[pl.ds(base + cq * ch, ch)],
                    sem_w[bq])
        for b in range(nbuf):
            if descs_w[b] is not None:
                descs_w[b].wait()

    return k(ys, slot0, slot1)


def kernel(hidden_states, gate_w, e_score_correction_bias, w1, w2, w3):
    bsz, seq_len, hidden_dim = hidden_states.shape
    x = hidden_states.reshape(T, D)

    sw, rowwt, cnt = _router_dispatch(x, gate_w, e_score_correction_bias)

    slot2 = sw[:, :2].astype(jnp.int32)
    slot0 = slot2[:, 0]
    slot1 = slot2[:, 1]
    wt_col = rowwt[:, 1:2]

    cnt_e = cnt[0, :E].astype(jnp.int32)
    nblk_e = (cnt_e + BT - 1) // BT
    ends = jnp.cumsum(nblk_e)
    nblk_total = ends[E - 1:E]
    bidx = jnp.arange(NBLK, dtype=jnp.int32)
    block_expert = jnp.minimum(
        jnp.sum((bidx[:, None] >= ends[None, :]).astype(jnp.int32), axis=1),
        E - 1)

    xs = _sc_dispatch(x, slot2)

    ys = _ffn(xs, wt_col, w1, w2, w3, block_expert, nblk_total)

    out = _sc_combine(ys, slot0, slot1)

    return out.reshape(bsz, seq_len, hidden_dim)
